# Initial kernel scaffold; baseline (speedup 1.0000x reference)
import jax
import jax.numpy as jnp
from jax.experimental import pallas as pl


def kernel(x, edge_index, W_in, b_in, A_in, B_in, gW1, gb1, lW1, lb1, lA1, lB1, g1, be1, gW2, gb2, lW2, lb2, lA2, lB2, g2, be2, W_out, b_out, A_out, B_out):
    raise NotImplementedError("write your pallas kernel here")


# speedup vs baseline: 1.0644x; 1.0644x over previous
import math

import jax
import jax.numpy as jnp
import numpy as np
from jax.experimental import pallas as pl
from jax.experimental.pallas import tpu as pltpu

_N, _E, _D, _R = 10000, 320000, 128, 4
_SCALING = 8.0 / 4.0


def _pe_full_table():
    pe = np.zeros((2000, _D), dtype=np.float32)
    position = np.arange(0, 2000, dtype=np.float32)[:, None]
    div_term = np.exp(
        np.arange(0, _D, 2, dtype=np.float32) * (-math.log(10000.0) / _D))
    pe[:, 0::2] = np.sin(position * div_term)
    pe[:, 1::2] = np.cos(position * div_term[: _D // 2])
    full = np.concatenate(
        [pe, np.broadcast_to(pe[1999], (_N - 2000, _D))], axis=0)
    return jnp.asarray(full)


_PE_FULL = _pe_full_table()


def _prologue_body(x_ref, pe_ref, w_ref, b_ref, a_ref, bb_ref, invo_ref,
                   h_ref, hs_ref):
    w_eff = w_ref[...] + _SCALING * jnp.dot(
        bb_ref[...], a_ref[...], preferred_element_type=jnp.float32)
    h = jnp.dot(x_ref[...], w_eff.T, preferred_element_type=jnp.float32)
    h = h + b_ref[...][None, :] + pe_ref[...]
    h_ref[...] = h
    hs_ref[...] = h * invo_ref[...]


def _layer_body(is_last, agg_ref, h_ref, hs_ref, invi_ref, invo_ref,
                gw_ref, gb_ref, lw_ref, lb_ref, la_ref, lbb_ref,
                g_ref, be_ref, wo_ref, bo_ref, ao_ref, bbo_ref,
                out0_ref, out1_ref=None):
    agg = (agg_ref[...] + hs_ref[...]) * invi_ref[...]
    lw_eff = lw_ref[...] + _SCALING * jnp.dot(
        lbb_ref[...], la_ref[...], preferred_element_type=jnp.float32)
    hn = jnp.dot(agg, gw_ref[...].T, preferred_element_type=jnp.float32)
    hn = hn + gb_ref[...][None, :]
    hn = hn + jnp.dot(h_ref[...], lw_eff.T,
                      preferred_element_type=jnp.float32) + lb_ref[...][None, :]
    mu = jnp.mean(hn, axis=-1, keepdims=True)
    var = jnp.mean((hn - mu) ** 2, axis=-1, keepdims=True)
    hn = (hn - mu) * jax.lax.rsqrt(var + 1e-5)
    hn = hn * g_ref[...][None, :] + be_ref[...][None, :]
    hn = 0.5 * hn * (1.0 + jax.lax.erf(hn * (1.0 / np.sqrt(2.0))))
    h_next = h_ref[...] + hn
    if is_last:
        wo_eff = wo_ref[...] + _SCALING * jnp.dot(
            bbo_ref[...], ao_ref[...], preferred_element_type=jnp.float32)
        out0_ref[...] = jnp.dot(
            h_next, wo_eff.T,
            preferred_element_type=jnp.float32) + bo_ref[...][None, :]
    else:
        out0_ref[...] = h_next
        out1_ref[...] = h_next * invo_ref[...]


def _tc_prologue(x, pe, W_in, b_in, A_in, B_in, inv_out):
    return pl.pallas_call(
        _prologue_body,
        out_shape=(
            jax.ShapeDtypeStruct((_N, _D), jnp.float32),
            jax.ShapeDtypeStruct((_N, _D), jnp.float32),
        ),
    )(x, pe, W_in, b_in, A_in, B_in, inv_out)


def _tc_layer(agg, h, hs, inv_in, inv_out, gW, gb, lW, lb, lA, lB, g, be):
    import functools
    return pl.pallas_call(
        functools.partial(_layer_body, False),
        out_shape=(
            jax.ShapeDtypeStruct((_N, _D), jnp.float32),
            jax.ShapeDtypeStruct((_N, _D), jnp.float32),
        ),
    )(agg, h, hs, inv_in, inv_out, gW, gb, lW, lb, lA, lB, g, be,
      gW, gb, lA, lB)


def _tc_layer_last(agg, h, hs, inv_in, gW, gb, lW, lb, lA, lB, g, be,
                   W_out, b_out, A_out, B_out):
    import functools
    return pl.pallas_call(
        functools.partial(_layer_body, True),
        out_shape=jax.ShapeDtypeStruct((_N, _D), jnp.float32),
    )(agg, h, hs, inv_in, inv_in, gW, gb, lW, lb, lA, lB, g, be,
      W_out, b_out, A_out, B_out)


def kernel(x, edge_index, W_in, b_in, A_in, B_in, gW1, gb1, lW1, lb1, lA1,
           lB1, g1, be1, gW2, gb2, lW2, lb2, lA2, lB2, g2, be2, W_out, b_out,
           A_out, B_out):
    src = edge_index[0]
    dst = edge_index[1]
    keep = (src != dst).astype(jnp.float32)

    deg_out = jnp.zeros((_N,), jnp.float32).at[src].add(keep) + 1.0
    deg_in = jnp.zeros((_N,), jnp.float32).at[dst].add(keep) + 1.0
    inv_out = (deg_out ** -0.5)[:, None]
    inv_in = (deg_in ** -0.5)[:, None]

    h, hs = _tc_prologue(x, _PE_FULL, W_in, b_in, A_in, B_in, inv_out)

    agg1 = jnp.zeros((_N, _D), jnp.float32).at[dst].add(hs[src] * keep[:, None])
    h, hs = _tc_layer(agg1, h, hs, inv_in, inv_out,
                      gW1, gb1, lW1, lb1, lA1, lB1, g1, be1)

    agg2 = jnp.zeros((_N, _D), jnp.float32).at[dst].add(hs[src] * keep[:, None])
    out = _tc_layer_last(agg2, h, hs, inv_in, gW2, gb2, lW2, lb2, lA2, lB2,
                         g2, be2, W_out, b_out, A_out, B_out)
    return out


# speedup vs baseline: 5.8302x; 5.4775x over previous
import functools
import math

import jax
import jax.numpy as jnp
import numpy as np
from jax.experimental import pallas as pl
from jax.experimental.pallas import tpu as pltpu
from jax.experimental.pallas import tpu_sc as plsc

_N, _E, _D, _R = 10000, 320000, 128, 4
_SCALING = 8.0 / 4.0

_NC, _NS, _L = 2, 16, 16
_NW = _NC * _NS
_ET = _E // _NW
_EC = -(-_ET // 128)
_ETP = _EC * 128
_TRASH = _N
_NPAD = 10240
_ZROWS = _NPAD // _NS
_OROWS = _N // _NS


def _pe_full_table():
    pe = np.zeros((2000, _D), dtype=np.float32)
    position = np.arange(0, 2000, dtype=np.float32)[:, None]
    div_term = np.exp(
        np.arange(0, _D, 2, dtype=np.float32) * (-math.log(10000.0) / _D))
    pe[:, 0::2] = np.sin(position * div_term)
    pe[:, 1::2] = np.cos(position * div_term[: _D // 2])
    return np.ascontiguousarray(np.concatenate(
        [pe, np.broadcast_to(pe[1999], (_N - 2000, _D))], axis=0))


_PE_FULL = _pe_full_table()


def _sc_mesh():
    return plsc.VectorSubcoreMesh(
        core_axis_name="c", subcore_axis_name="s",
        num_cores=_NC, num_subcores=_NS)


def _munge_body(src_hbm, dst_hbm, srcp_hbm, dstp_hbm, dego_hbm, degi_hbm,
                src_v, dst_v, dego_v, degi_v, dstp_v):
    c = jax.lax.axis_index("c")
    s = jax.lax.axis_index("s")
    wid = c * _NS + s
    base = wid * _ET
    pltpu.sync_copy(src_hbm.at[pl.ds(base, _ET)], src_v.at[pl.ds(0, _ET)])
    pltpu.sync_copy(dst_hbm.at[pl.ds(base, _ET)], dst_v.at[pl.ds(0, _ET)])
    izero = jnp.zeros((_L,), jnp.int32)
    fzero = jnp.zeros((_L,), jnp.float32)
    trash = jnp.full((_L,), _TRASH, jnp.int32)
    for j in range((_ETP - _ET) // _L):
        src_v[pl.ds(_ET + j * _L, _L)] = izero
        dst_v[pl.ds(_ET + j * _L, _L)] = izero

    def zbody(i, carry):
        dego_v[pl.ds(i * _L, _L)] = fzero
        degi_v[pl.ds(i * _L, _L)] = fzero
        return carry
    jax.lax.fori_loop(0, _N // _L, zbody, 0)

    def mbody(r, carry):
        for j in range(8):
            off = r * 128 + j * _L
            s16 = src_v[pl.ds(off, _L)]
            d16 = dst_v[pl.ds(off, _L)]
            keep = s16 != d16
            keepf = jnp.where(keep, 1.0, 0.0).astype(jnp.float32)
            plsc.addupdate_scatter(dego_v, [s16], keepf)
            plsc.addupdate_scatter(degi_v, [d16], keepf)
            dstp_v[r, pl.ds(j * _L, _L)] = jnp.where(keep, d16, trash)
        return carry
    jax.lax.fori_loop(0, _EC, mbody, 0)

    pltpu.sync_copy(src_v, srcp_hbm.at[wid])
    pltpu.sync_copy(dstp_v, dstp_hbm.at[wid])
    pltpu.sync_copy(dego_v, dego_hbm.at[wid])
    pltpu.sync_copy(degi_v, degi_hbm.at[wid])


@functools.cache
def _sc_munge():
    return pl.kernel(
        _munge_body,
        out_type=(
            jax.ShapeDtypeStruct((_NW, _ETP), jnp.int32),
            jax.ShapeDtypeStruct((_NW, _EC, 128), jnp.int32),
            jax.ShapeDtypeStruct((_NW, _N), jnp.float32),
            jax.ShapeDtypeStruct((_NW, _N), jnp.float32),
        ),
        mesh=_sc_mesh(),
        compiler_params=pltpu.CompilerParams(needs_layout_passes=False),
        scratch_types=[
            pltpu.VMEM((_ETP,), jnp.int32),
            pltpu.VMEM((_ETP,), jnp.int32),
            pltpu.VMEM((_N,), jnp.float32),
            pltpu.VMEM((_N,), jnp.float32),
            pltpu.VMEM((_EC, 128), jnp.int32),
        ],
    )


def _edge_body(hs_hbm, srcp_hbm, dstp_hbm, out_hbm,
               src_v, dst_v, rows_v, acc_shr, sem):
    c = jax.lax.axis_index("c")
    s = jax.lax.axis_index("s")
    wid = c * _NS + s
    fzero = jnp.zeros((_L,), jnp.float32)

    def zbody(r, carry):
        for j in range(8):
            rows_v[r, pl.ds(j * _L, _L)] = fzero
        return carry
    jax.lax.fori_loop(0, 128, zbody, 0)
    for k in range(_ZROWS // 128):
        pltpu.sync_copy(rows_v, acc_shr.at[pl.ds(s * _ZROWS + k * 128, 128)])
    pltpu.sync_copy(srcp_hbm.at[wid], src_v)
    pltpu.sync_copy(dstp_hbm.at[wid], dst_v)
    plsc.subcore_barrier()

    def cbody(g, carry):
        pltpu.async_copy(
            hs_hbm.at[src_v.at[pl.ds(g * 128, 128)]], rows_v, sem).wait()
        pltpu.sync_copy(rows_v, acc_shr.at[dst_v.at[g]], add=True)
        return carry
    jax.lax.fori_loop(0, _EC, cbody, 0)
    plsc.subcore_barrier()
    pltpu.sync_copy(acc_shr.at[pl.ds(s * _ZROWS, _ZROWS)],
                    out_hbm.at[c, pl.ds(s * _ZROWS, _ZROWS)])


@functools.cache
def _sc_edge():
    return pl.kernel(
        _edge_body,
        out_type=jax.ShapeDtypeStruct((_NC, _NPAD, _D), jnp.float32),
        mesh=_sc_mesh(),
        compiler_params=pltpu.CompilerParams(needs_layout_passes=False),
        scratch_types=[
            pltpu.VMEM((_ETP,), jnp.int32),
            pltpu.VMEM((_EC, 128), jnp.int32),
            pltpu.VMEM((128, _D), jnp.float32),
            pltpu.VMEM_SHARED((_NPAD, _D), jnp.float32),
            pltpu.SemaphoreType.DMA,
        ],
    )


def _prologue_body(x_ref, pe_ref, w_ref, b_ref, a_ref, bb_ref,
                   dego_ref, degi_ref, h_ref, hs_ref, invi_ref, invo_ref):
    w_eff = w_ref[...] + _SCALING * jnp.dot(
        bb_ref[...], a_ref[...], preferred_element_type=jnp.float32)
    h = jnp.dot(x_ref[...], w_eff.T, preferred_element_type=jnp.float32)
    h = h + b_ref[...][None, :] + pe_ref[...]
    invo = jax.lax.rsqrt(jnp.sum(dego_ref[...], axis=0) + 1.0)[:, None]
    invi = jax.lax.rsqrt(jnp.sum(degi_ref[...], axis=0) + 1.0)[:, None]
    h_ref[...] = h
    hs_ref[...] = h * invo
    invi_ref[...] = invi
    invo_ref[...] = invo


def _tc_prologue(x, pe, W_in, b_in, A_in, B_in, dego, degi):
    return pl.pallas_call(
        _prologue_body,
        out_shape=(
            jax.ShapeDtypeStruct((_N, _D), jnp.float32),
            jax.ShapeDtypeStruct((_N, _D), jnp.float32),
            jax.ShapeDtypeStruct((_N, 1), jnp.float32),
            jax.ShapeDtypeStruct((_N, 1), jnp.float32),
        ),
    )(x, pe, W_in, b_in, A_in, B_in, dego, degi)


def _layer_body(is_last, agg_ref, h_ref, hs_ref, invi_ref, invo_ref,
                gw_ref, gb_ref, lw_ref, lb_ref, la_ref, lbb_ref,
                g_ref, be_ref, wo_ref, bo_ref, ao_ref, bbo_ref,
                out0_ref, out1_ref=None):
    agg = (agg_ref[0, :_N] + agg_ref[1, :_N] + hs_ref[...]) * invi_ref[...]
    lw_eff = lw_ref[...] + _SCALING * jnp.dot(
        lbb_ref[...], la_ref[...], preferred_element_type=jnp.float32)
    hn = jnp.dot(agg, gw_ref[...].T, preferred_element_type=jnp.float32)
    hn = hn + gb_ref[...][None, :]
    hn = hn + jnp.dot(h_ref[...], lw_eff.T,
                      preferred_element_type=jnp.float32) + lb_ref[...][None, :]
    mu = jnp.mean(hn, axis=-1, keepdims=True)
    var = jnp.mean((hn - mu) ** 2, axis=-1, keepdims=True)
    hn = (hn - mu) * jax.lax.rsqrt(var + 1e-5)
    hn = hn * g_ref[...][None, :] + be_ref[...][None, :]
    hn = 0.5 * hn * (1.0 + jax.lax.erf(hn * (1.0 / np.sqrt(2.0))))
    h_next = h_ref[...] + hn
    if is_last:
        wo_eff = wo_ref[...] + _SCALING * jnp.dot(
            bbo_ref[...], ao_ref[...], preferred_element_type=jnp.float32)
        out0_ref[...] = jnp.dot(
            h_next, wo_eff.T,
            preferred_element_type=jnp.float32) + bo_ref[...][None, :]
    else:
        out0_ref[...] = h_next
        out1_ref[...] = h_next * invo_ref[...]


def _tc_layer(agg, h, hs, inv_in, inv_out, gW, gb, lW, lb, lA, lB, g, be):
    return pl.pallas_call(
        functools.partial(_layer_body, False),
        out_shape=(
            jax.ShapeDtypeStruct((_N, _D), jnp.float32),
            jax.ShapeDtypeStruct((_N, _D), jnp.float32),
        ),
    )(agg, h, hs, inv_in, inv_out, gW, gb, lW, lb, lA, lB, g, be,
      gW, gb, lA, lB)


def _tc_layer_last(agg, h, hs, inv_in, gW, gb, lW, lb, lA, lB, g, be,
                   W_out, b_out, A_out, B_out):
    return pl.pallas_call(
        functools.partial(_layer_body, True),
        out_shape=jax.ShapeDtypeStruct((_N, _D), jnp.float32),
    )(agg, h, hs, inv_in, inv_in, gW, gb, lW, lb, lA, lB, g, be,
      W_out, b_out, A_out, B_out)


def kernel(x, edge_index, W_in, b_in, A_in, B_in, gW1, gb1, lW1, lb1, lA1,
           lB1, g1, be1, gW2, gb2, lW2, lb2, lA2, lB2, g2, be2, W_out, b_out,
           A_out, B_out):
    src = edge_index[0]
    dst = edge_index[1]

    srcp, dstp, dego, degi = _sc_munge()(src, dst)
    h, hs, inv_in, inv_out = _tc_prologue(
        x, _PE_FULL, W_in, b_in, A_in, B_in, dego, degi)

    aggp = _sc_edge()(hs, srcp, dstp)
    h, hs = _tc_layer(aggp, h, hs, inv_in, inv_out,
                      gW1, gb1, lW1, lb1, lA1, lB1, g1, be1)

    aggp = _sc_edge()(hs, srcp, dstp)
    out = _tc_layer_last(aggp, h, hs, inv_in, gW2, gb2, lW2, lb2, lA2, lB2,
                         g2, be2, W_out, b_out, A_out, B_out)
    return out


# speedup vs baseline: 6.6775x; 1.1453x over previous
import functools
import math

import jax
import jax.numpy as jnp
import numpy as np
from jax.experimental import pallas as pl
from jax.experimental.pallas import tpu as pltpu
from jax.experimental.pallas import tpu_sc as plsc

_N, _E, _D, _R = 10000, 320000, 128, 4
_SCALING = 8.0 / 4.0

_NC, _NS, _L = 2, 16, 16
_NW = _NC * _NS
_ET = _E // _NW
_EC = -(-_ET // 128)
_ETP = _EC * 128
_CH = 64
_EC2 = _ETP // _CH
_TRASH = _N
_NPAD = 10240
_ZROWS = _NPAD // _NS
_OROWS = _N // _NS


def _pe_full_table():
    pe = np.zeros((2000, _D), dtype=np.float32)
    position = np.arange(0, 2000, dtype=np.float32)[:, None]
    div_term = np.exp(
        np.arange(0, _D, 2, dtype=np.float32) * (-math.log(10000.0) / _D))
    pe[:, 0::2] = np.sin(position * div_term)
    pe[:, 1::2] = np.cos(position * div_term[: _D // 2])
    return np.ascontiguousarray(np.concatenate(
        [pe, np.broadcast_to(pe[1999], (_N - 2000, _D))], axis=0))


_PE_FULL = _pe_full_table()


def _sc_mesh():
    return plsc.VectorSubcoreMesh(
        core_axis_name="c", subcore_axis_name="s",
        num_cores=_NC, num_subcores=_NS)


def _munge_body(src_hbm, dst_hbm, srcp_hbm, dstp_hbm, dego_hbm, degi_hbm,
                src_v, dst_v, dego_v, degi_v, dstp_v):
    c = jax.lax.axis_index("c")
    s = jax.lax.axis_index("s")
    wid = c * _NS + s
    base = wid * _ET
    pltpu.sync_copy(src_hbm.at[pl.ds(base, _ET)], src_v.at[pl.ds(0, _ET)])
    pltpu.sync_copy(dst_hbm.at[pl.ds(base, _ET)], dst_v.at[pl.ds(0, _ET)])
    izero = jnp.zeros((_L,), jnp.int32)
    fzero = jnp.zeros((_L,), jnp.float32)
    trash = jnp.full((_L,), _TRASH, jnp.int32)
    for j in range((_ETP - _ET) // _L):
        src_v[pl.ds(_ET + j * _L, _L)] = izero
        dst_v[pl.ds(_ET + j * _L, _L)] = izero

    def zbody(i, carry):
        dego_v[pl.ds(i * _L, _L)] = fzero
        degi_v[pl.ds(i * _L, _L)] = fzero
        return carry
    jax.lax.fori_loop(0, _N // _L, zbody, 0)

    def mbody(r, carry):
        for j in range(8):
            off = r * 128 + j * _L
            s16 = src_v[pl.ds(off, _L)]
            d16 = dst_v[pl.ds(off, _L)]
            keep = s16 != d16
            keepf = jnp.where(keep, 1.0, 0.0).astype(jnp.float32)
            plsc.addupdate_scatter(dego_v, [s16], keepf)
            plsc.addupdate_scatter(degi_v, [d16], keepf)
            dstp_v[r * 2 + j // 4, pl.ds((j % 4) * _L, _L)] = (
                jnp.where(keep, d16, trash))
        return carry
    jax.lax.fori_loop(0, _EC, mbody, 0)

    pltpu.sync_copy(src_v, srcp_hbm.at[wid])
    pltpu.sync_copy(dstp_v, dstp_hbm.at[wid])
    pltpu.sync_copy(dego_v, dego_hbm.at[wid])
    pltpu.sync_copy(degi_v, degi_hbm.at[wid])


@functools.cache
def _sc_munge():
    return pl.kernel(
        _munge_body,
        out_type=(
            jax.ShapeDtypeStruct((_NW, _ETP), jnp.int32),
            jax.ShapeDtypeStruct((_NW, _EC2, _CH), jnp.int32),
            jax.ShapeDtypeStruct((_NW, _N), jnp.float32),
            jax.ShapeDtypeStruct((_NW, _N), jnp.float32),
        ),
        mesh=_sc_mesh(),
        compiler_params=pltpu.CompilerParams(needs_layout_passes=False),
        scratch_types=[
            pltpu.VMEM((_ETP,), jnp.int32),
            pltpu.VMEM((_ETP,), jnp.int32),
            pltpu.VMEM((_N,), jnp.float32),
            pltpu.VMEM((_N,), jnp.float32),
            pltpu.VMEM((_EC2, _CH), jnp.int32),
        ],
    )


def _edge_body(hs_hbm, srcp_hbm, dstp_hbm, out_hbm,
               src_v, dst_v, rows0, rows1, acc_shr, gsem):
    c = jax.lax.axis_index("c")
    s = jax.lax.axis_index("s")
    wid = c * _NS + s
    fzero = jnp.zeros((_L,), jnp.float32)

    def zbody(r, carry):
        for j in range(8):
            rows0[r, pl.ds(j * _L, _L)] = fzero
        return carry
    jax.lax.fori_loop(0, _CH, zbody, 0)
    for k in range(_ZROWS // _CH):
        pltpu.sync_copy(rows0, acc_shr.at[pl.ds(s * _ZROWS + k * _CH, _CH)])
    pltpu.sync_copy(srcp_hbm.at[wid], src_v)
    pltpu.sync_copy(dstp_hbm.at[wid], dst_v)
    plsc.subcore_barrier()

    bufs = (rows0, rows1)

    def _gather(ch):
        return hs_hbm.at[src_v.at[pl.ds(ch * _CH, _CH)]]

    pltpu.async_copy(_gather(0), rows0, gsem)
    npair = (_EC2 - 2) // 2

    def cbody(i, carry):
        g = i * 2
        for par in range(2):
            ch = g + par
            pltpu.async_copy(_gather(ch + 1), bufs[1 - par], gsem)
            pltpu.make_async_copy(_gather(ch), bufs[par], gsem).wait()
            pltpu.sync_copy(bufs[par], acc_shr.at[dst_v.at[ch]], add=True)
        return carry
    jax.lax.fori_loop(0, npair, cbody, 0)
    for ch in range(npair * 2, _EC2):
        if ch + 1 < _EC2:
            pltpu.async_copy(_gather(ch + 1), bufs[(ch + 1) % 2], gsem)
        pltpu.make_async_copy(_gather(ch), bufs[ch % 2], gsem).wait()
        pltpu.sync_copy(bufs[ch % 2], acc_shr.at[dst_v.at[ch]], add=True)
    plsc.subcore_barrier()
    pltpu.sync_copy(acc_shr.at[pl.ds(s * _ZROWS, _ZROWS)],
                    out_hbm.at[c, pl.ds(s * _ZROWS, _ZROWS)])


@functools.cache
def _sc_edge():
    return pl.kernel(
        _edge_body,
        out_type=jax.ShapeDtypeStruct((_NC, _NPAD, _D), jnp.float32),
        mesh=_sc_mesh(),
        compiler_params=pltpu.CompilerParams(needs_layout_passes=False),
        scratch_types=[
            pltpu.VMEM((_ETP,), jnp.int32),
            pltpu.VMEM((_EC2, _CH), jnp.int32),
            pltpu.VMEM((_CH, _D), jnp.float32),
            pltpu.VMEM((_CH, _D), jnp.float32),
            pltpu.VMEM_SHARED((_NPAD, _D), jnp.float32),
            pltpu.SemaphoreType.DMA,
        ],
    )


def _prologue_body(x_ref, pe_ref, w_ref, b_ref, a_ref, bb_ref,
                   dego_ref, degi_ref, h_ref, hs_ref, invi_ref, invo_ref):
    w_eff = w_ref[...] + _SCALING * jnp.dot(
        bb_ref[...], a_ref[...], preferred_element_type=jnp.float32)
    h = jnp.dot(x_ref[...], w_eff.T, preferred_element_type=jnp.float32)
    h = h + b_ref[...][None, :] + pe_ref[...]
    invo = jax.lax.rsqrt(jnp.sum(dego_ref[...], axis=0) + 1.0)[:, None]
    invi = jax.lax.rsqrt(jnp.sum(degi_ref[...], axis=0) + 1.0)[:, None]
    h_ref[...] = h
    hs_ref[...] = h * invo
    invi_ref[...] = invi
    invo_ref[...] = invo


def _tc_prologue(x, pe, W_in, b_in, A_in, B_in, dego, degi):
    return pl.pallas_call(
        _prologue_body,
        out_shape=(
            jax.ShapeDtypeStruct((_N, _D), jnp.float32),
            jax.ShapeDtypeStruct((_N, _D), jnp.float32),
            jax.ShapeDtypeStruct((_N, 1), jnp.float32),
            jax.ShapeDtypeStruct((_N, 1), jnp.float32),
        ),
    )(x, pe, W_in, b_in, A_in, B_in, dego, degi)


def _layer_body(is_last, agg_ref, h_ref, hs_ref, invi_ref, invo_ref,
                gw_ref, gb_ref, lw_ref, lb_ref, la_ref, lbb_ref,
                g_ref, be_ref, wo_ref, bo_ref, ao_ref, bbo_ref,
                out0_ref, out1_ref=None):
    agg = (agg_ref[0, :_N] + agg_ref[1, :_N] + hs_ref[...]) * invi_ref[...]
    lw_eff = lw_ref[...] + _SCALING * jnp.dot(
        lbb_ref[...], la_ref[...], preferred_element_type=jnp.float32)
    hn = jnp.dot(agg, gw_ref[...].T, preferred_element_type=jnp.float32)
    hn = hn + gb_ref[...][None, :]
    hn = hn + jnp.dot(h_ref[...], lw_eff.T,
                      preferred_element_type=jnp.float32) + lb_ref[...][None, :]
    mu = jnp.mean(hn, axis=-1, keepdims=True)
    var = jnp.mean((hn - mu) ** 2, axis=-1, keepdims=True)
    hn = (hn - mu) * jax.lax.rsqrt(var + 1e-5)
    hn = hn * g_ref[...][None, :] + be_ref[...][None, :]
    hn = 0.5 * hn * (1.0 + jax.lax.erf(hn * (1.0 / np.sqrt(2.0))))
    h_next = h_ref[...] + hn
    if is_last:
        wo_eff = wo_ref[...] + _SCALING * jnp.dot(
            bbo_ref[...], ao_ref[...], preferred_element_type=jnp.float32)
        out0_ref[...] = jnp.dot(
            h_next, wo_eff.T,
            preferred_element_type=jnp.float32) + bo_ref[...][None, :]
    else:
        out0_ref[...] = h_next
        out1_ref[...] = h_next * invo_ref[...]


def _tc_layer(agg, h, hs, inv_in, inv_out, gW, gb, lW, lb, lA, lB, g, be):
    return pl.pallas_call(
        functools.partial(_layer_body, False),
        out_shape=(
            jax.ShapeDtypeStruct((_N, _D), jnp.float32),
            jax.ShapeDtypeStruct((_N, _D), jnp.float32),
        ),
    )(agg, h, hs, inv_in, inv_out, gW, gb, lW, lb, lA, lB, g, be,
      gW, gb, lA, lB)


def _tc_layer_last(agg, h, hs, inv_in, gW, gb, lW, lb, lA, lB, g, be,
                   W_out, b_out, A_out, B_out):
    return pl.pallas_call(
        functools.partial(_layer_body, True),
        out_shape=jax.ShapeDtypeStruct((_N, _D), jnp.float32),
    )(agg, h, hs, inv_in, inv_in, gW, gb, lW, lb, lA, lB, g, be,
      W_out, b_out, A_out, B_out)


def kernel(x, edge_index, W_in, b_in, A_in, B_in, gW1, gb1, lW1, lb1, lA1,
           lB1, g1, be1, gW2, gb2, lW2, lb2, lA2, lB2, g2, be2, W_out, b_out,
           A_out, B_out):
    src = edge_index[0]
    dst = edge_index[1]

    srcp, dstp, dego, degi = _sc_munge()(src, dst)
    h, hs, inv_in, inv_out = _tc_prologue(
        x, _PE_FULL, W_in, b_in, A_in, B_in, dego, degi)

    aggp = _sc_edge()(hs, srcp, dstp)
    h, hs = _tc_layer(aggp, h, hs, inv_in, inv_out,
                      gW1, gb1, lW1, lb1, lA1, lB1, g1, be1)

    aggp = _sc_edge()(hs, srcp, dstp)
    out = _tc_layer_last(aggp, h, hs, inv_in, gW2, gb2, lW2, lb2, lA2, lB2,
                         g2, be2, W_out, b_out, A_out, B_out)
    return out


# speedup vs baseline: 7.3417x; 1.0995x over previous
import functools
import math

import jax
import jax.numpy as jnp
import numpy as np
from jax.experimental import pallas as pl
from jax.experimental.pallas import tpu as pltpu
from jax.experimental.pallas import tpu_sc as plsc

_N, _E, _D, _R = 10000, 320000, 128, 4
_SCALING = 8.0 / 4.0

_NC, _NS, _L = 2, 16, 16
_NW = _NC * _NS
_ET = _E // _NW
_EC = -(-_ET // 128)
_ETP = _EC * 128
_CH = 64
_EC2 = _ETP // _CH
_TRASH = _N
_NPAD = 10240
_ZROWS = _NPAD // _NS
_OROWS = _N // _NS


def _pe_full_table():
    pe = np.zeros((2000, _D), dtype=np.float32)
    position = np.arange(0, 2000, dtype=np.float32)[:, None]
    div_term = np.exp(
        np.arange(0, _D, 2, dtype=np.float32) * (-math.log(10000.0) / _D))
    pe[:, 0::2] = np.sin(position * div_term)
    pe[:, 1::2] = np.cos(position * div_term[: _D // 2])
    return np.ascontiguousarray(np.concatenate(
        [pe, np.broadcast_to(pe[1999], (_N - 2000, _D))], axis=0))


_PE_FULL = _pe_full_table()


def _sc_mesh():
    return plsc.VectorSubcoreMesh(
        core_axis_name="c", subcore_axis_name="s",
        num_cores=_NC, num_subcores=_NS)


def _munge_body(src_hbm, dst_hbm, srcp_hbm, dstp_hbm, dego_hbm, degi_hbm,
                src_v, dst_v, dego_v, degi_v, dstp_v):
    c = jax.lax.axis_index("c")
    s = jax.lax.axis_index("s")
    wid = c * _NS + s
    base = wid * _ET
    pltpu.sync_copy(src_hbm.at[pl.ds(base, _ET)], src_v.at[pl.ds(0, _ET)])
    pltpu.sync_copy(dst_hbm.at[pl.ds(base, _ET)], dst_v.at[pl.ds(0, _ET)])
    izero = jnp.zeros((_L,), jnp.int32)
    fzero = jnp.zeros((_L,), jnp.float32)
    trash = jnp.full((_L,), _TRASH, jnp.int32)
    for j in range((_ETP - _ET) // _L):
        src_v[pl.ds(_ET + j * _L, _L)] = izero
        dst_v[pl.ds(_ET + j * _L, _L)] = izero

    def zbody(i, carry):
        dego_v[pl.ds(i * _L, _L)] = fzero
        degi_v[pl.ds(i * _L, _L)] = fzero
        return carry
    jax.lax.fori_loop(0, _N // _L, zbody, 0)

    def mbody(r, carry):
        for j in range(8):
            off = r * 128 + j * _L
            s16 = src_v[pl.ds(off, _L)]
            d16 = dst_v[pl.ds(off, _L)]
            keep = s16 != d16
            keepf = jnp.where(keep, 1.0, 0.0).astype(jnp.float32)
            plsc.addupdate_scatter(dego_v, [s16], keepf)
            plsc.addupdate_scatter(degi_v, [d16], keepf)
            dstp_v[r * 2 + j // 4, pl.ds((j % 4) * _L, _L)] = (
                jnp.where(keep, d16, trash))
        return carry
    jax.lax.fori_loop(0, _EC, mbody, 0)

    pltpu.sync_copy(src_v, srcp_hbm.at[wid])
    pltpu.sync_copy(dstp_v, dstp_hbm.at[wid])
    pltpu.sync_copy(dego_v, dego_hbm.at[wid])
    pltpu.sync_copy(degi_v, degi_hbm.at[wid])


@functools.cache
def _sc_munge():
    return pl.kernel(
        _munge_body,
        out_type=(
            jax.ShapeDtypeStruct((_NW, _ETP), jnp.int32),
            jax.ShapeDtypeStruct((_NW, _EC2, _CH), jnp.int32),
            jax.ShapeDtypeStruct((_NW, _N), jnp.float32),
            jax.ShapeDtypeStruct((_NW, _N), jnp.float32),
        ),
        mesh=_sc_mesh(),
        compiler_params=pltpu.CompilerParams(needs_layout_passes=False),
        scratch_types=[
            pltpu.VMEM((_ETP,), jnp.int32),
            pltpu.VMEM((_ETP,), jnp.int32),
            pltpu.VMEM((_N,), jnp.float32),
            pltpu.VMEM((_N,), jnp.float32),
            pltpu.VMEM((_EC2, _CH), jnp.int32),
        ],
    )


def _edge_body(hs_hbm, srcp_hbm, dstp_hbm, out_hbm,
               src_v, dstst, rows0, rows1, rows2, acc_shr, gsem, isem):
    c = jax.lax.axis_index("c")
    s = jax.lax.axis_index("s")
    wid = c * _NS + s
    fzero = jnp.zeros((_L,), jnp.float32)

    def zbody(r, carry):
        for j in range(8):
            rows0[r, pl.ds(j * _L, _L)] = fzero
        return carry
    jax.lax.fori_loop(0, _CH, zbody, 0)
    for k in range(_ZROWS // _CH):
        pltpu.sync_copy(rows0, acc_shr.at[pl.ds(s * _ZROWS + k * _CH, _CH)])
    pltpu.sync_copy(srcp_hbm.at[wid], src_v)
    plsc.subcore_barrier()

    bufs = (rows0, rows1, rows2)
    nbuf = len(bufs)

    def _gather(ch):
        return hs_hbm.at[src_v.at[pl.ds(ch * _CH, _CH)]]

    def _istage(ch, slot):
        return (dstp_hbm.at[wid, ch], dstst.at[slot])

    def _start(ch, slot):
        pltpu.async_copy(*_istage(ch, slot), isem)
        pltpu.async_copy(_gather(ch), bufs[slot], gsem)

    def _finish(ch, slot):
        pltpu.make_async_copy(_gather(ch), bufs[slot], gsem).wait()
        pltpu.make_async_copy(*_istage(ch, slot), isem).wait()
        pltpu.sync_copy(bufs[slot], acc_shr.at[dstst.at[slot]], add=True)

    for p in range(nbuf - 1):
        _start(p, p)
    nloop = (_EC2 - (nbuf - 1)) // nbuf

    def cbody(i, carry):
        g = i * nbuf
        for par in range(nbuf):
            ch = g + par
            _start(ch + nbuf - 1, (par + nbuf - 1) % nbuf)
            _finish(ch, par)
        return carry
    jax.lax.fori_loop(0, nloop, cbody, 0)
    for ch in range(nloop * nbuf, _EC2):
        if ch + nbuf - 1 < _EC2:
            _start(ch + nbuf - 1, (ch + nbuf - 1) % nbuf)
        _finish(ch, ch % nbuf)
    plsc.subcore_barrier()
    pltpu.sync_copy(acc_shr.at[pl.ds(s * _ZROWS, _ZROWS)],
                    out_hbm.at[c, pl.ds(s * _ZROWS, _ZROWS)])


@functools.cache
def _sc_edge():
    return pl.kernel(
        _edge_body,
        out_type=jax.ShapeDtypeStruct((_NC, _NPAD, _D), jnp.float32),
        mesh=_sc_mesh(),
        compiler_params=pltpu.CompilerParams(needs_layout_passes=False),
        scratch_types=[
            pltpu.VMEM((_ETP,), jnp.int32),
            pltpu.VMEM((3, _CH), jnp.int32),
            pltpu.VMEM((_CH, _D), jnp.float32),
            pltpu.VMEM((_CH, _D), jnp.float32),
            pltpu.VMEM((_CH, _D), jnp.float32),
            pltpu.VMEM_SHARED((_NPAD, _D), jnp.float32),
            pltpu.SemaphoreType.DMA,
            pltpu.SemaphoreType.DMA,
        ],
    )


def _prologue_body(x_ref, pe_ref, w_ref, b_ref, a_ref, bb_ref,
                   dego_ref, degi_ref, h_ref, hs_ref, invi_ref, invo_ref):
    w_eff = w_ref[...] + _SCALING * jnp.dot(
        bb_ref[...], a_ref[...], preferred_element_type=jnp.float32)
    h = jnp.dot(x_ref[...], w_eff.T, preferred_element_type=jnp.float32)
    h = h + b_ref[...][None, :] + pe_ref[...]
    invo = jax.lax.rsqrt(jnp.sum(dego_ref[...], axis=0) + 1.0)[:, None]
    invi = jax.lax.rsqrt(jnp.sum(degi_ref[...], axis=0) + 1.0)[:, None]
    h_ref[...] = h
    hs_ref[...] = h * invo
    invi_ref[...] = invi
    invo_ref[...] = invo


def _tc_prologue(x, pe, W_in, b_in, A_in, B_in, dego, degi):
    return pl.pallas_call(
        _prologue_body,
        out_shape=(
            jax.ShapeDtypeStruct((_N, _D), jnp.float32),
            jax.ShapeDtypeStruct((_N, _D), jnp.float32),
            jax.ShapeDtypeStruct((_N, 1), jnp.float32),
            jax.ShapeDtypeStruct((_N, 1), jnp.float32),
        ),
    )(x, pe, W_in, b_in, A_in, B_in, dego, degi)


def _layer_body(is_last, agg_ref, h_ref, hs_ref, invi_ref, invo_ref,
                gw_ref, gb_ref, lw_ref, lb_ref, la_ref, lbb_ref,
                g_ref, be_ref, wo_ref, bo_ref, ao_ref, bbo_ref,
                out0_ref, out1_ref=None):
    agg = (agg_ref[0, :_N] + agg_ref[1, :_N] + hs_ref[...]) * invi_ref[...]
    lw_eff = lw_ref[...] + _SCALING * jnp.dot(
        lbb_ref[...], la_ref[...], preferred_element_type=jnp.float32)
    hn = jnp.dot(agg, gw_ref[...].T, preferred_element_type=jnp.float32)
    hn = hn + gb_ref[...][None, :]
    hn = hn + jnp.dot(h_ref[...], lw_eff.T,
                      preferred_element_type=jnp.float32) + lb_ref[...][None, :]
    mu = jnp.mean(hn, axis=-1, keepdims=True)
    var = jnp.mean((hn - mu) ** 2, axis=-1, keepdims=True)
    hn = (hn - mu) * jax.lax.rsqrt(var + 1e-5)
    hn = hn * g_ref[...][None, :] + be_ref[...][None, :]
    hn = 0.5 * hn * (1.0 + jax.lax.erf(hn * (1.0 / np.sqrt(2.0))))
    h_next = h_ref[...] + hn
    if is_last:
        wo_eff = wo_ref[...] + _SCALING * jnp.dot(
            bbo_ref[...], ao_ref[...], preferred_element_type=jnp.float32)
        out0_ref[...] = jnp.dot(
            h_next, wo_eff.T,
            preferred_element_type=jnp.float32) + bo_ref[...][None, :]
    else:
        out0_ref[...] = h_next
        out1_ref[...] = h_next * invo_ref[...]


def _tc_layer(agg, h, hs, inv_in, inv_out, gW, gb, lW, lb, lA, lB, g, be):
    return pl.pallas_call(
        functools.partial(_layer_body, False),
        out_shape=(
            jax.ShapeDtypeStruct((_N, _D), jnp.float32),
            jax.ShapeDtypeStruct((_N, _D), jnp.float32),
        ),
    )(agg, h, hs, inv_in, inv_out, gW, gb, lW, lb, lA, lB, g, be,
      gW, gb, lA, lB)


def _tc_layer_last(agg, h, hs, inv_in, gW, gb, lW, lb, lA, lB, g, be,
                   W_out, b_out, A_out, B_out):
    return pl.pallas_call(
        functools.partial(_layer_body, True),
        out_shape=jax.ShapeDtypeStruct((_N, _D), jnp.float32),
    )(agg, h, hs, inv_in, inv_in, gW, gb, lW, lb, lA, lB, g, be,
      W_out, b_out, A_out, B_out)


def kernel(x, edge_index, W_in, b_in, A_in, B_in, gW1, gb1, lW1, lb1, lA1,
           lB1, g1, be1, gW2, gb2, lW2, lb2, lA2, lB2, g2, be2, W_out, b_out,
           A_out, B_out):
    src = edge_index[0]
    dst = edge_index[1]

    srcp, dstp, dego, degi = _sc_munge()(src, dst)
    h, hs, inv_in, inv_out = _tc_prologue(
        x, _PE_FULL, W_in, b_in, A_in, B_in, dego, degi)

    aggp = _sc_edge()(hs, srcp, dstp)
    h, hs = _tc_layer(aggp, h, hs, inv_in, inv_out,
                      gW1, gb1, lW1, lb1, lA1, lB1, g1, be1)

    aggp = _sc_edge()(hs, srcp, dstp)
    out = _tc_layer_last(aggp, h, hs, inv_in, gW2, gb2, lW2, lb2, lA2, lB2,
                         g2, be2, W_out, b_out, A_out, B_out)
    return out
